# Initial kernel scaffold; baseline (speedup 1.0000x reference)
import jax
import jax.numpy as jnp
from jax.experimental import pallas as pl


def kernel(proj_vars, res_state, rows, cols, vals):
    raise NotImplementedError("write your pallas kernel here")


# speedup vs baseline: 157.0703x; 157.0703x over previous
import functools

import jax
import jax.numpy as jnp
from jax import lax
from jax.experimental import pallas as pl
from jax.experimental.pallas import tpu as pltpu
from jax.experimental.pallas import tpu_sc as plsc

_RES = 16384
_LEAK = 0.6
_BIAS = 1.6
_NC = 2
_NS = 16
_NW = _NC * _NS
_CHUNK = 4096


def _sc_partials(rows, cols, vals, res_state, per_w):
    n_chunks = per_w // _CHUNK
    mesh = plsc.VectorSubcoreMesh(core_axis_name="c", subcore_axis_name="s")

    @functools.partial(
        pl.kernel,
        out_type=jax.ShapeDtypeStruct((_NW, _RES), jnp.float32),
        mesh=mesh,
        scratch_types=[
            pltpu.VMEM((_RES,), jnp.float32),
            pltpu.VMEM((_RES,), jnp.float32),
            pltpu.VMEM((_CHUNK,), jnp.int32),
            pltpu.VMEM((_CHUNK,), jnp.int32),
            pltpu.VMEM((_CHUNK,), jnp.float32),
        ],
        compiler_params=pltpu.CompilerParams(needs_layout_passes=False),
    )
    def body(rows_h, cols_h, vals_h, res_h, out_h, res_v, acc_v, rbuf, cbuf, vbuf):
        wid = lax.axis_index("s") * _NC + lax.axis_index("c")
        base = wid * per_w
        pltpu.sync_copy(res_h, res_v)

        zeros = jnp.zeros((16,), jnp.float32)

        def zbody(i, _):
            acc_v[pl.ds(i * 16, 16)] = zeros
            return ()

        lax.fori_loop(0, _RES // 16, zbody, ())

        def cbody(ci, _):
            off = base + ci * _CHUNK
            pltpu.sync_copy(rows_h.at[pl.ds(off, _CHUNK)], rbuf)
            pltpu.sync_copy(cols_h.at[pl.ds(off, _CHUNK)], cbuf)
            pltpu.sync_copy(vals_h.at[pl.ds(off, _CHUNK)], vbuf)

            def vbody(j, _):
                sl = pl.ds(j * 16, 16)
                g = plsc.load_gather(res_v, [cbuf[sl]])
                plsc.addupdate_scatter(acc_v, [rbuf[sl]], vbuf[sl] * g)
                return ()

            lax.fori_loop(0, _CHUNK // 16, vbody, ())
            return ()

        lax.fori_loop(0, n_chunks, cbody, ())
        pltpu.sync_copy(acc_v, out_h.at[wid])

    return body(rows, cols, vals, res_state)


def _reduce_epilogue(partials, proj_vars, res_state):
    def body(p_ref, pv_ref, rs_ref, o_ref):
        s = jnp.sum(p_ref[...], axis=0)
        act = jnp.tanh(s + pv_ref[...] + _BIAS)
        o_ref[...] = _LEAK * act + (1.0 - _LEAK) * rs_ref[...]

    return pl.pallas_call(
        body,
        out_shape=jax.ShapeDtypeStruct((_RES,), jnp.float32),
    )(partials, proj_vars, res_state)


def kernel(proj_vars, res_state, rows, cols, vals):
    nnz = rows.shape[0]
    per_w = -(-nnz // (_NW * _CHUNK)) * _CHUNK
    pad = _NW * per_w - nnz
    rows_p = jnp.pad(rows.astype(jnp.int32), (0, pad))
    cols_p = jnp.pad(cols.astype(jnp.int32), (0, pad))
    vals_p = jnp.pad(vals.astype(jnp.float32), (0, pad))
    res32 = res_state.astype(jnp.float32)
    partials = _sc_partials(rows_p, cols_p, vals_p, res32, per_w)
    return _reduce_epilogue(partials, proj_vars.astype(jnp.float32), res32)


# speedup vs baseline: 232.3892x; 1.4795x over previous
import functools

import jax
import jax.numpy as jnp
from jax import lax
from jax.experimental import pallas as pl
from jax.experimental.pallas import tpu as pltpu
from jax.experimental.pallas import tpu_sc as plsc

_RES = 16384
_LEAK = 0.6
_BIAS = 1.6
_NC = 2
_NS = 16
_NW = _NC * _NS
_CHUNK = 4096


def _sc_partials(rows, cols, vals, res_state, per_w):
    n_chunks = per_w // _CHUNK
    mesh = plsc.VectorSubcoreMesh(core_axis_name="c", subcore_axis_name="s")

    @functools.partial(
        pl.kernel,
        out_type=jax.ShapeDtypeStruct((_NW, _RES), jnp.float32),
        mesh=mesh,
        scratch_types=[
            pltpu.VMEM((_RES,), jnp.float32),
            pltpu.VMEM((_RES,), jnp.float32),
            pltpu.VMEM((2, _CHUNK), jnp.int32),
            pltpu.VMEM((2, _CHUNK), jnp.int32),
            pltpu.VMEM((2, _CHUNK), jnp.float32),
            pltpu.SemaphoreType.DMA((2,)),
            pltpu.SemaphoreType.DMA,
        ],
        compiler_params=pltpu.CompilerParams(needs_layout_passes=False),
    )
    def body(rows_h, cols_h, vals_h, res_h, out_h,
             res_v, acc_v, rbuf, cbuf, vbuf, sems, res_sem):
        wid = lax.axis_index("s") * _NC + lax.axis_index("c")
        base = wid * per_w

        def issue(ci, slot):
            off = base + ci * _CHUNK
            pltpu.async_copy(rows_h.at[pl.ds(off, _CHUNK)], rbuf.at[slot], sems.at[slot])
            pltpu.async_copy(cols_h.at[pl.ds(off, _CHUNK)], cbuf.at[slot], sems.at[slot])
            pltpu.async_copy(vals_h.at[pl.ds(off, _CHUNK)], vbuf.at[slot], sems.at[slot])

        def drain(slot):
            pltpu.make_async_copy(rows_h.at[pl.ds(0, _CHUNK)], rbuf.at[slot], sems.at[slot]).wait()
            pltpu.make_async_copy(cols_h.at[pl.ds(0, _CHUNK)], cbuf.at[slot], sems.at[slot]).wait()
            pltpu.make_async_copy(vals_h.at[pl.ds(0, _CHUNK)], vbuf.at[slot], sems.at[slot]).wait()

        issue(0, 0)
        res_copy = pltpu.async_copy(res_h, res_v, res_sem)

        zeros = jnp.zeros((16,), jnp.float32)

        @plsc.parallel_loop(0, _RES // 16, unroll=8)
        def _(i):
            acc_v[pl.ds(i * 16, 16)] = zeros

        res_copy.wait()

        def cbody(ci, _):
            slot = lax.rem(ci, 2)

            @pl.when(ci + 1 < n_chunks)
            def _():
                issue(ci + 1, 1 - slot)

            drain(slot)

            @plsc.parallel_loop(0, _CHUNK // 16, unroll=8)
            def _(j):
                sl = pl.ds(j * 16, 16)
                g = plsc.load_gather(res_v, [cbuf[slot, sl]])
                plsc.addupdate_scatter(acc_v, [rbuf[slot, sl]], vbuf[slot, sl] * g)

            return ()

        lax.fori_loop(0, n_chunks, cbody, ())
        pltpu.sync_copy(acc_v, out_h.at[wid])

    return body(rows, cols, vals, res_state)


def _reduce_epilogue(partials, proj_vars, res_state):
    def body(p_ref, pv_ref, rs_ref, o_ref):
        s = jnp.sum(p_ref[...], axis=0)
        act = jnp.tanh(s + pv_ref[...] + _BIAS)
        o_ref[...] = _LEAK * act + (1.0 - _LEAK) * rs_ref[...]

    return pl.pallas_call(
        body,
        out_shape=jax.ShapeDtypeStruct((_RES,), jnp.float32),
    )(partials, proj_vars, res_state)


def kernel(proj_vars, res_state, rows, cols, vals):
    nnz = rows.shape[0]
    per_w = -(-nnz // (_NW * _CHUNK)) * _CHUNK
    pad = _NW * per_w - nnz
    rows_p = jnp.pad(rows.astype(jnp.int32), (0, pad))
    cols_p = jnp.pad(cols.astype(jnp.int32), (0, pad))
    vals_p = jnp.pad(vals.astype(jnp.float32), (0, pad))
    res32 = res_state.astype(jnp.float32)
    partials = _sc_partials(rows_p, cols_p, vals_p, res32, per_w)
    return _reduce_epilogue(partials, proj_vars.astype(jnp.float32), res32)
